# Initial kernel scaffold; baseline (speedup 1.0000x reference)
import jax
import jax.numpy as jnp
from jax.experimental import pallas as pl


def kernel(x, edge_index, edge_type, W1_rel, W1_root, alpha1, proj1_w, proj1_b, W2_rel, W2_root, alpha2, proj2_w, proj2_b, out_w, out_b):
    raise NotImplementedError("write your pallas kernel here")


# speedup vs baseline: 6.7236x; 6.7236x over previous
import functools

import jax
import jax.numpy as jnp
from jax import lax
from jax.experimental import pallas as pl
from jax.experimental.pallas import tpu as pltpu
from jax.experimental.pallas import tpu_sc as plsc

N = 10000
NP = 10240
D = 128
NRL = 4
E = 320000
NC = 2
NS = 16
NW = NC * NS
B = 128
CH = 79
EPAD = NW * CH * B
RPT = NP // NS
RB = 1024
GRID = NP // RB


def _seg_body(with_cnt, *refs):
    if with_cnt:
        (table, gidx, dst, zrow, zcnt, p_out, cnt_out,
         gidx_v, dst_v, rows_v, acc_sh, sem, ones_v, cnt_sh) = refs
    else:
        (table, gidx, dst, zrow, p_out,
         gidx_v, dst_v, rows_v, acc_sh, sem) = refs
    c = lax.axis_index("c")
    s = lax.axis_index("s")
    w = c * NS + s

    pltpu.sync_copy(zrow, acc_sh.at[pl.ds(s * RPT, RPT)])
    pltpu.sync_copy(gidx.at[w], gidx_v)
    pltpu.sync_copy(dst.at[w], dst_v)
    if with_cnt:
        pltpu.sync_copy(zcnt, cnt_sh.at[pl.ds(s * RPT, RPT)])
        for k in range(B // 16):
            ones_v[pl.ds(k * 16, 16)] = jnp.ones((16,), jnp.float32)
    plsc.subcore_barrier()

    def chunk(j, carry):
        pltpu.async_copy(table.at[gidx_v.at[j]], rows_v, sem).wait()
        pltpu.sync_copy(rows_v, acc_sh.at[dst_v.at[j]], add=True)
        if with_cnt:
            pltpu.sync_copy(ones_v, cnt_sh.at[dst_v.at[j]], add=True)
        return carry

    lax.fori_loop(0, CH, chunk, 0)
    plsc.subcore_barrier()

    base = c * NP + s * RPT
    pltpu.sync_copy(acc_sh.at[pl.ds(s * RPT, RPT)], p_out.at[pl.ds(base, RPT)])
    if with_cnt:
        pltpu.sync_copy(cnt_sh.at[pl.ds(s * RPT, RPT)],
                        cnt_out.at[pl.ds(base, RPT)])


def _make_seg(with_cnt):
    outs = [jax.ShapeDtypeStruct((NC * NP, D), jnp.float32)]
    scratch = [
        pltpu.VMEM((CH, B), jnp.int32),
        pltpu.VMEM((CH, B), jnp.int32),
        pltpu.VMEM((B, D), jnp.float32),
        pltpu.VMEM_SHARED((NP, D), jnp.float32),
        pltpu.SemaphoreType.DMA,
    ]
    if with_cnt:
        outs.append(jax.ShapeDtypeStruct((NC * NP,), jnp.float32))
        scratch += [
            pltpu.VMEM((B,), jnp.float32),
            pltpu.VMEM_SHARED((NP,), jnp.float32),
        ]
    mesh = plsc.VectorSubcoreMesh(core_axis_name="c", subcore_axis_name="s",
                                  num_cores=NC, num_subcores=NS)
    return pl.kernel(functools.partial(_seg_body, with_cnt),
                     out_type=outs, mesh=mesh, scratch_types=scratch)


@functools.lru_cache(maxsize=None)
def _get_seg(with_cnt):
    return _make_seg(with_cnt)


def _seg_cnt(table, gidx3, dst3, zrow, zcnt):
    return _get_seg(True)(table, gidx3, dst3, zrow, zcnt)


def _seg(table, gidx3, dst3, zrow):
    return _get_seg(False)(table, gidx3, dst3, zrow)[0]


def _gidx_body(t_ref, s_ref, o_ref):
    o_ref[...] = t_ref[...] * NP + s_ref[...]


_k_gidx = pl.pallas_call(
    _gidx_body,
    out_shape=jax.ShapeDtypeStruct((NW * CH, B), jnp.int32),
)


def _transform_body(x_ref, w_ref, o_ref):
    o_ref[0] = jnp.dot(x_ref[...], w_ref[0],
                       preferred_element_type=jnp.float32)


_k_transform = pl.pallas_call(
    _transform_body,
    grid=(NRL + 1, GRID),
    in_specs=[
        pl.BlockSpec((RB, D), lambda r, j: (j, 0)),
        pl.BlockSpec((1, D, D), lambda r, j: (r, 0, 0)),
    ],
    out_specs=pl.BlockSpec((1, RB, D), lambda r, j: (r, j, 0)),
    out_shape=jax.ShapeDtypeStruct((NRL + 1, NP, D), jnp.float32),
)


def _invdeg_body(c_ref, o_ref):
    d = c_ref[0, :] + c_ref[1, :]
    o_ref[0, :] = 1.0 / jnp.maximum(d, 1.0)


_k_invdeg = pl.pallas_call(
    _invdeg_body,
    out_shape=jax.ShapeDtypeStruct((1, NP), jnp.float32),
)


def _postr_body(y_ref, p0_ref, p1_ref, i_ref, o_ref):
    o_ref[...] = jnp.maximum(
        (y_ref[0] + p0_ref[...] + p1_ref[...]) * i_ref[...], 0.0)


_k_post_rgcn = pl.pallas_call(
    _postr_body,
    grid=(GRID,),
    in_specs=[
        pl.BlockSpec((1, RB, D), lambda j: (NRL, j, 0)),
        pl.BlockSpec((RB, D), lambda j: (j, 0)),
        pl.BlockSpec((RB, D), lambda j: (j + GRID, 0)),
        pl.BlockSpec((RB, 1), lambda j: (j, 0)),
    ],
    out_specs=pl.BlockSpec((RB, D), lambda j: (j, 0)),
    out_shape=jax.ShapeDtypeStruct((NP, D), jnp.float32),
)


def _postge_body(h_ref, p0_ref, p1_ref, i_ref, pw_ref, pb_ref, a_ref, o_ref):
    agg = (p0_ref[...] + p1_ref[...]) * i_ref[...]
    v = lax.dot_general(agg, pw_ref[...], (((1,), (1,)), ((), ())),
                        preferred_element_type=jnp.float32)
    o_ref[...] = h_ref[...] + a_ref[...] * (v + pb_ref[...])


_k_post_ge = pl.pallas_call(
    _postge_body,
    grid=(GRID,),
    in_specs=[
        pl.BlockSpec((RB, D), lambda j: (j, 0)),
        pl.BlockSpec((RB, D), lambda j: (j, 0)),
        pl.BlockSpec((RB, D), lambda j: (j + GRID, 0)),
        pl.BlockSpec((RB, 1), lambda j: (j, 0)),
        pl.BlockSpec((D, D), lambda j: (0, 0)),
        pl.BlockSpec((1, D), lambda j: (0, 0)),
        pl.BlockSpec((1, 1), lambda j: (0, 0)),
    ],
    out_specs=pl.BlockSpec((RB, D), lambda j: (j, 0)),
    out_shape=jax.ShapeDtypeStruct((NP, D), jnp.float32),
)


def _final_body(h_ref, w_ref, b_ref, o_ref):
    o_ref[...] = lax.dot_general(
        h_ref[...], w_ref[...], (((1,), (1,)), ((), ())),
        preferred_element_type=jnp.float32) + b_ref[...]


_k_final = pl.pallas_call(
    _final_body,
    grid=(GRID,),
    in_specs=[
        pl.BlockSpec((RB, D), lambda j: (j, 0)),
        pl.BlockSpec((D, D), lambda j: (0, 0)),
        pl.BlockSpec((1, D), lambda j: (0, 0)),
    ],
    out_specs=pl.BlockSpec((RB, D), lambda j: (j, 0)),
    out_shape=jax.ShapeDtypeStruct((NP, D), jnp.float32),
)


def kernel(x, edge_index, edge_type, W1_rel, W1_root, alpha1, proj1_w,
           proj1_b, W2_rel, W2_root, alpha2, proj2_w, proj2_b, out_w, out_b):
    f32 = jnp.float32
    src = edge_index[0].astype(jnp.int32)
    dst = edge_index[1].astype(jnp.int32)
    typ = edge_type.astype(jnp.int32)
    pad = EPAD - src.shape[0]
    src2 = jnp.pad(src, (0, pad)).reshape(NW * CH, B)
    dst2 = jnp.pad(dst, (0, pad), constant_values=N + 100).reshape(NW * CH, B)
    typ2 = jnp.pad(typ, (0, pad)).reshape(NW * CH, B)

    gidx3 = _k_gidx(typ2, src2).reshape(NW, CH, B)
    src3 = src2.reshape(NW, CH, B)
    dst3 = dst2.reshape(NW, CH, B)

    x_pad = jnp.pad(x, ((0, NP - N), (0, 0)))
    zrow = jnp.zeros((RPT, D), f32)
    zcnt = jnp.zeros((RPT,), f32)

    wcat1 = jnp.concatenate([W1_rel, W1_root[None]], axis=0)
    wcat2 = jnp.concatenate([W2_rel, W2_root[None]], axis=0)
    pb1 = proj1_b.reshape(1, D)
    pb2 = proj2_b.reshape(1, D)
    ob = out_b.reshape(1, D)
    a1 = jnp.asarray(alpha1, f32).reshape(1, 1)
    a2 = jnp.asarray(alpha2, f32).reshape(1, 1)

    y1 = _k_transform(x_pad, wcat1)
    p1, cnt = _seg_cnt(y1.reshape((NRL + 1) * NP, D), gidx3, dst3, zrow, zcnt)
    inv = _k_invdeg(cnt.reshape(NC, NP)).reshape(NP, 1)
    h = _k_post_rgcn(y1, p1, p1, inv)
    p2 = _seg(h, src3, dst3, zrow)
    h = _k_post_ge(h, p2, p2, inv, proj1_w, pb1, a1)
    y2 = _k_transform(h, wcat2)
    p3 = _seg(y2.reshape((NRL + 1) * NP, D), gidx3, dst3, zrow)
    h = _k_post_rgcn(y2, p3, p3, inv)
    p4 = _seg(h, src3, dst3, zrow)
    h = _k_post_ge(h, p4, p4, inv, proj2_w, pb2, a2)

    out = _k_final(h, out_w, ob)
    return out[:N]
